# Initial kernel scaffold; baseline (speedup 1.0000x reference)
import jax
import jax.numpy as jnp
from jax.experimental import pallas as pl


def kernel(x, edge_index, Wp, bp, Wl, bl, Wr, gamma, beta):
    raise NotImplementedError("write your pallas kernel here")


# speedup vs baseline: 7.1901x; 7.1901x over previous
import functools

import jax
import jax.numpy as jnp
from jax import lax
from jax.experimental import pallas as pl
from jax.experimental.pallas import tpu as pltpu
from jax.experimental.pallas import tpu_sc as plsc

_N = 10000
_E = 320000
_D = 128
_NC = 2
_NS = 16
_NW = _NC * _NS
_K = 80
_C = _E // (_NW * _K)
_NPAD = 10240
_ROWS_PER_TILE = _NPAD // _NS


def _h_tc(x, Wp, bp):
    def body(x_ref, w_ref, b_ref, o_ref):
        acc = lax.dot_general(
            x_ref[...], w_ref[...],
            dimension_numbers=(((1,), (1,)), ((), ())),
            preferred_element_type=jnp.float32)
        o_ref[...] = jnp.maximum(acc + b_ref[...], 0.0)

    return pl.pallas_call(
        body,
        grid=(10,),
        in_specs=[
            pl.BlockSpec((_N // 10, _D), lambda i: (i, 0)),
            pl.BlockSpec((_D, _D), lambda i: (0, 0)),
            pl.BlockSpec((1, _D), lambda i: (0, 0)),
        ],
        out_specs=pl.BlockSpec((_N // 10, _D), lambda i: (i, 0)),
        out_shape=jax.ShapeDtypeStruct((_N, _D), jnp.float32),
    )(x, Wp, bp.reshape(1, _D))


def _agg_sc(src3, dst3, h, zrows):
    mesh = plsc.VectorSubcoreMesh(core_axis_name="c", subcore_axis_name="s")

    @functools.partial(
        pl.kernel,
        mesh=mesh,
        out_type=jax.ShapeDtypeStruct((_NC * _NPAD, _D), jnp.float32),
        scratch_types=[
            pltpu.VMEM((_C, _K), jnp.int32),
            pltpu.VMEM((_C, _K), jnp.int32),
            pltpu.VMEM((_K, _D), jnp.float32),
            pltpu.VMEM_SHARED((_NPAD, _D), jnp.float32),
            pltpu.SemaphoreType.DMA,
        ],
    )
    def k(src_hbm, dst_hbm, h_hbm, z_hbm, out_hbm,
          src_v, dst_v, rows_v, acc_sh, gsem):
        cid = lax.axis_index("c")
        sid = lax.axis_index("s")
        w = cid * _NS + sid

        pltpu.sync_copy(z_hbm, acc_sh.at[pl.ds(sid * _ROWS_PER_TILE,
                                               _ROWS_PER_TILE)])
        pltpu.sync_copy(src_hbm.at[w], src_v)
        pltpu.sync_copy(dst_hbm.at[w], dst_v)
        plsc.subcore_barrier()

        def chunk(j, carry):
            pltpu.async_copy(h_hbm.at[src_v.at[j]], rows_v, gsem).wait()
            pltpu.sync_copy(rows_v, acc_sh.at[dst_v.at[j]], add=True)
            return carry

        lax.fori_loop(0, _C, chunk, 0)

        plsc.subcore_barrier()
        pltpu.sync_copy(
            acc_sh.at[pl.ds(sid * _ROWS_PER_TILE, _ROWS_PER_TILE)],
            out_hbm.at[pl.ds(cid * _NPAD + sid * _ROWS_PER_TILE,
                             _ROWS_PER_TILE)])

    return k(src3, dst3, h, zrows)


def _out_tc(p0, p1, x, Wl, bl, Wr, gamma, beta):
    def body(p0_ref, p1_ref, x_ref, wl_ref, bl_ref, wr_ref, g_ref, b_ref,
             o_ref):
        agg = p0_ref[...] + p1_ref[...]
        out = lax.dot_general(
            agg, wl_ref[...],
            dimension_numbers=(((1,), (1,)), ((), ())),
            preferred_element_type=jnp.float32)
        out = out + bl_ref[...]
        out = out + lax.dot_general(
            x_ref[...], wr_ref[...],
            dimension_numbers=(((1,), (1,)), ((), ())),
            preferred_element_type=jnp.float32)
        mean = jnp.mean(out, axis=1, keepdims=True)
        cent = out - mean
        var = jnp.mean(cent * cent, axis=1, keepdims=True)
        o_ref[...] = cent * lax.rsqrt(var + 1e-5) * g_ref[...] + b_ref[...]

    blk = _N // 10
    row_spec = pl.BlockSpec((blk, _D), lambda i: (i, 0))
    full_spec = pl.BlockSpec((_D, _D), lambda i: (0, 0))
    vec_spec = pl.BlockSpec((1, _D), lambda i: (0, 0))
    return pl.pallas_call(
        body,
        grid=(10,),
        in_specs=[row_spec, row_spec, row_spec, full_spec, vec_spec,
                  full_spec, vec_spec, vec_spec],
        out_specs=row_spec,
        out_shape=jax.ShapeDtypeStruct((_N, _D), jnp.float32),
    )(p0, p1, x, Wl, bl.reshape(1, _D), Wr, gamma.reshape(1, _D),
      beta.reshape(1, _D))


def kernel(x, edge_index, Wp, bp, Wl, bl, Wr, gamma, beta):
    h = _h_tc(x, Wp, bp)
    src3 = edge_index[0].reshape(_NW, _C, _K)
    dst3 = edge_index[1].reshape(_NW, _C, _K)
    zrows = jnp.zeros((_ROWS_PER_TILE, _D), jnp.float32)
    partials = _agg_sc(src3, dst3, h, zrows)
    return _out_tc(partials[:_N], partials[_NPAD:_NPAD + _N], x, Wl, bl, Wr,
                   gamma, beta)
